# baseline (speedup 1.0000x reference)
import functools

import jax
import jax.numpy as jnp
from jax import lax
from jax.experimental import pallas as pl
from jax.experimental.pallas import tpu as pltpu
from jax.experimental.pallas import tpu_sc as plsc

_H = 2048
_HP = _H // 2
_I = 1408
_E = 8
_K = 2
_T = 4096
_NSL = _T * _K
_BM = 256
_NPAD = _NSL + _E * _BM
_NB = _NPAD // _BM
_NC = 2
_NSC = 16
_NW = _NC * _NSC
_TPW = _T // _NW
_CG = 16
_CGD = 32


def _sc_mesh():
    return plsc.VectorSubcoreMesh(
        core_axis_name="c", subcore_axis_name="s",
        num_cores=_NC, num_subcores=_NSC)


def _tc_pack_body(x_ref, o_ref):
    u = lax.bitcast_convert_type(x_ref[...], jnp.uint32)
    b = (u + jnp.uint32(0x7FFF) + ((u >> 16) & jnp.uint32(1))) >> 16
    packed = b[:, :_HP] | (b[:, _HP:] << 16)
    o_ref[...] = lax.bitcast_convert_type(packed, jnp.int32)


def _tc_pack(x):
    return pl.pallas_call(
        _tc_pack_body,
        grid=(8,),
        in_specs=[pl.BlockSpec((_T // 8, _H), lambda i: (i, 0))],
        out_specs=pl.BlockSpec((_T // 8, _HP), lambda i: (i, 0)),
        out_shape=jax.ShapeDtypeStruct((_T, _HP), jnp.int32),
    )(x)


def _tc_route_body(e_ref, pos_ref, be_ref):
    ids = e_ref[...]
    dn = (((1,), (0,)), ((), ()))
    up_tri = (lax.broadcasted_iota(jnp.int32, (128, 128), 0)
              <= lax.broadcasted_iota(jnp.int32, (128, 128), 1)
              ).astype(jnp.float32)
    lo_tri = (lax.broadcasted_iota(jnp.int32, (64, 64), 0)
              > lax.broadcasted_iota(jnp.int32, (64, 64), 1)
              ).astype(jnp.float32)
    masks = []
    ranks = []
    counts = []
    for e in range(_E):
        m = (ids == e).astype(jnp.float32)
        p = lax.dot_general(m, up_tri, dn, preferred_element_type=jnp.float32)
        t = p[:, 127:128]
        rows_before = lax.dot_general(
            lo_tri, t, dn, preferred_element_type=jnp.float32)
        counts.append(rows_before[63:64, :] + t[63:64, :])
        masks.append(m)
        ranks.append(p + rows_before - 1.0)
    pos = jnp.zeros((64, 128), jnp.float32)
    off = jnp.zeros((1, 1), jnp.float32)
    ends = []
    for e in range(_E):
        padded = jnp.floor((counts[e] + (_BM - 1)) / _BM) * _BM
        pos = pos + masks[e] * (off + ranks[e])
        off = off + padded
        ends.append(off)
    pos_ref[...] = pos.astype(jnp.int32)
    blk = lax.broadcasted_iota(
        jnp.int32, (8, 128), 1).astype(jnp.float32) * _BM
    be = jnp.zeros((8, 128), jnp.float32)
    for e in range(_E):
        be = be + (blk >= ends[e]).astype(jnp.float32)
    be_ref[...] = jnp.minimum(be, _E - 1).astype(jnp.int32)


def _tc_route(e2d):
    return pl.pallas_call(
        _tc_route_body,
        grid=(1,),
        in_specs=[pl.BlockSpec((64, 128), lambda i: (0, 0))],
        out_specs=(pl.BlockSpec((64, 128), lambda i: (0, 0)),
                   pl.BlockSpec((8, 128), lambda i: (0, 0))),
        out_shape=(jax.ShapeDtypeStruct((64, 128), jnp.int32),
                   jax.ShapeDtypeStruct((8, 128), jnp.int32)),
    )(e2d)


def _sc_dispatch(x_packed, pos_sm, w_sm):
    n_chunks = _TPW // _CGD

    @functools.partial(
        pl.kernel,
        mesh=_sc_mesh(),
        out_type=(
            jax.ShapeDtypeStruct((_NPAD, _HP), jnp.int32),
            jax.ShapeDtypeStruct((_NPAD,), jnp.float32),
        ),
        scratch_types=[
            pltpu.VMEM((_CGD,), jnp.int32),
            pltpu.VMEM((_CGD,), jnp.int32),
            pltpu.VMEM((_CGD,), jnp.int32),
            pltpu.VMEM((_CGD,), jnp.int32),
            pltpu.VMEM((_CGD,), jnp.float32),
            pltpu.VMEM((_CGD,), jnp.float32),
            pltpu.VMEM((_CGD,), jnp.float32),
            pltpu.VMEM((_CGD,), jnp.float32),
            pltpu.VMEM((_CGD, _HP), jnp.int32),
            pltpu.VMEM((_CGD, _HP), jnp.int32),
            pltpu.SemaphoreType.DMA,
            pltpu.SemaphoreType.DMA,
            pltpu.SemaphoreType.DMA,
            pltpu.SemaphoreType.DMA,
            pltpu.SemaphoreType.DMA,
            pltpu.SemaphoreType.DMA,
            pltpu.SemaphoreType.DMA,
            pltpu.SemaphoreType.DMA,
        ],
    )
    def k(xp_hbm, pos_hbm, w_hbm, xs_hbm, wp_hbm,
          p0a, p0b, p1a, p1b, w0a, w0b, w1a, w1b, xca, xcb,
          s0a, s0b, s1a, s1b, s2a, s2b, s3a, s3b):
        wid = lax.axis_index("s") * _NC + lax.axis_index("c")
        tok0 = wid * _TPW
        p0 = (p0a, p0b)
        p1 = (p1a, p1b)
        w0 = (w0a, w0b)
        w1 = (w1a, w1b)
        xc = (xca, xcb)
        sx0 = (s0a, s0b)
        sx1 = (s1a, s1b)
        sw0 = (s2a, s2b)
        sw1 = (s3a, s3b)
        hs = [None] * n_chunks
        for c in range(n_chunks):
            p = c & 1
            if c >= 2:
                for h in hs[c - 2]:
                    h.wait()
            off = tok0 + c * _CGD
            pltpu.sync_copy(pos_hbm.at[pl.ds(off, _CGD)], p0[p])
            pltpu.sync_copy(pos_hbm.at[pl.ds(_T + off, _CGD)], p1[p])
            pltpu.sync_copy(w_hbm.at[pl.ds(off, _CGD)], w0[p])
            pltpu.sync_copy(w_hbm.at[pl.ds(_T + off, _CGD)], w1[p])
            pltpu.sync_copy(xp_hbm.at[pl.ds(off, _CGD)], xc[p])
            hs[c] = (
                pltpu.async_copy(xc[p], xs_hbm.at[p0[p]], sx0[p]),
                pltpu.async_copy(xc[p], xs_hbm.at[p1[p]], sx1[p]),
                pltpu.async_copy(w0[p], wp_hbm.at[p0[p]], sw0[p]),
                pltpu.async_copy(w1[p], wp_hbm.at[p1[p]], sw1[p]),
            )
        for c in (n_chunks - 2, n_chunks - 1):
            for h in hs[c]:
                h.wait()

    return k(x_packed, pos_sm, w_sm)


def _tc_gemm_body(be_ref, x_ref, gpw_ref, upw_ref, dpw_ref, wp_ref, y_ref):
    del be_ref
    dn = (((1,), (1,)), ((), ()))
    xi = x_ref[...]
    lo = lax.bitcast_convert_type(xi << 16, jnp.float32).astype(jnp.bfloat16)
    hi = lax.bitcast_convert_type(
        xi & jnp.int32(-65536), jnp.float32).astype(jnp.bfloat16)
    gpw = gpw_ref[0]
    upw = upw_ref[0]
    f32 = jnp.float32
    gp = (lax.dot_general(lo, gpw[:, :_HP], dn, preferred_element_type=f32)
          + lax.dot_general(hi, gpw[:, _HP:], dn, preferred_element_type=f32))
    up = (lax.dot_general(lo, upw[:, :_HP], dn, preferred_element_type=f32)
          + lax.dot_general(hi, upw[:, _HP:], dn, preferred_element_type=f32))
    act = (gp * jax.nn.sigmoid(gp) * up).astype(jnp.bfloat16)
    y = lax.dot_general(act, dpw_ref[0], dn, preferred_element_type=f32)
    y_ref[...] = y * wp_ref[...]


def _tc_grouped_gemm(block_expert, x_sorted, gpw, upw, dpw, w_pad):
    grid_spec = pltpu.PrefetchScalarGridSpec(
        num_scalar_prefetch=1,
        grid=(_NB,),
        in_specs=[
            pl.BlockSpec((_BM, _HP), lambda i, be: (i, 0)),
            pl.BlockSpec((1, _I, _H), lambda i, be: (be[i], 0, 0)),
            pl.BlockSpec((1, _I, _H), lambda i, be: (be[i], 0, 0)),
            pl.BlockSpec((1, _H, _I), lambda i, be: (be[i], 0, 0)),
            pl.BlockSpec((_BM, 1), lambda i, be: (i, 0)),
        ],
        out_specs=pl.BlockSpec((_BM, _H), lambda i, be: (i, 0)),
    )
    return pl.pallas_call(
        _tc_gemm_body,
        grid_spec=grid_spec,
        out_shape=jax.ShapeDtypeStruct((_NPAD, _H), jnp.float32),
    )(block_expert, x_sorted, gpw, upw, dpw, w_pad)


def _sc_combine(y_sorted, pos_sm):
    @functools.partial(
        pl.kernel,
        mesh=_sc_mesh(),
        out_type=jax.ShapeDtypeStruct((_T, _H), jnp.float32),
        scratch_types=[
            pltpu.VMEM((_CG,), jnp.int32),
            pltpu.VMEM((_CG,), jnp.int32),
            pltpu.VMEM((_CG, _H), jnp.float32),
            pltpu.VMEM((_CG, _H), jnp.float32),
            pltpu.SemaphoreType.DMA,
            pltpu.SemaphoreType.DMA,
        ],
    )
    def k(y_hbm, pos_hbm, out_hbm, i0_v, i1_v, a_v, b_v, s0, s1):
        wid = lax.axis_index("s") * _NC + lax.axis_index("c")
        for c in range(_TPW // _CG):
            off = wid * _TPW + c * _CG
            pltpu.sync_copy(pos_hbm.at[pl.ds(off, _CG)], i0_v)
            pltpu.sync_copy(pos_hbm.at[pl.ds(_T + off, _CG)], i1_v)
            cp0 = pltpu.async_copy(y_hbm.at[i0_v], a_v, s0)
            cp1 = pltpu.async_copy(y_hbm.at[i1_v], b_v, s1)
            cp0.wait()
            cp1.wait()

            def add_col(j, carry):
                for r in range(_CG):
                    sl = pl.ds(j * 16, 16)
                    a_v[r, sl] = a_v[r, sl] + b_v[r, sl]
                return carry

            lax.fori_loop(0, _H // 16, add_col, 0)
            pltpu.sync_copy(a_v, out_hbm.at[pl.ds(off, _CG)])

    return k(y_sorted, pos_sm)


def kernel(hidden_states, gate_w, gate_proj_w, up_proj_w, down_proj_w):
    b, s, h = hidden_states.shape
    x = hidden_states.reshape(-1, h)

    router_logits = x @ gate_w.T
    routing_weights = jax.nn.softmax(router_logits.astype(jnp.float32), axis=1)
    rw_topk, selected_experts = jax.lax.top_k(routing_weights, _K)
    rw_topk = rw_topk / jnp.sum(rw_topk, axis=-1, keepdims=True)
    e_sm = selected_experts.astype(jnp.int32).T.reshape(64, 128)
    w_sm = rw_topk.T.reshape(_NSL)

    x_packed = _tc_pack(x)
    pos2d, be2d = _tc_route(e_sm)
    pos_sm = pos2d.reshape(_NSL)
    block_expert = be2d.reshape(-1)[:_NB]
    x_sorted, w_pad = _sc_dispatch(x_packed, pos_sm, w_sm)
    y_sorted = _tc_grouped_gemm(
        block_expert, x_sorted,
        gate_proj_w.astype(jnp.bfloat16),
        up_proj_w.astype(jnp.bfloat16),
        down_proj_w.astype(jnp.bfloat16),
        w_pad.reshape(_NPAD, 1))
    out = _sc_combine(y_sorted, pos_sm)
    return out.reshape(b, s, h)


# speedup vs baseline: 1.0289x; 1.0289x over previous
import functools

import jax
import jax.numpy as jnp
from jax import lax
from jax.experimental import pallas as pl
from jax.experimental.pallas import tpu as pltpu
from jax.experimental.pallas import tpu_sc as plsc

_H = 2048
_HP = _H // 2
_I = 1408
_E = 8
_K = 2
_T = 4096
_NSL = _T * _K
_BM = 256
_NPAD = _NSL + _E * _BM
_NB = _NPAD // _BM
_NC = 2
_NSC = 16
_NW = _NC * _NSC
_TPW = _T // _NW
_CG = 16
_CGD = 32


def _sc_mesh():
    return plsc.VectorSubcoreMesh(
        core_axis_name="c", subcore_axis_name="s",
        num_cores=_NC, num_subcores=_NSC)


def _tc_pack_body(x_ref, o_ref):
    u = lax.bitcast_convert_type(x_ref[...], jnp.uint32)
    b = (u + jnp.uint32(0x7FFF) + ((u >> 16) & jnp.uint32(1))) >> 16
    packed = b[:, :_HP] | (b[:, _HP:] << 16)
    o_ref[...] = lax.bitcast_convert_type(packed, jnp.int32)


def _tc_pack(x):
    return pl.pallas_call(
        _tc_pack_body,
        grid=(8,),
        in_specs=[pl.BlockSpec((_T // 8, _H), lambda i: (i, 0))],
        out_specs=pl.BlockSpec((_T // 8, _HP), lambda i: (i, 0)),
        out_shape=jax.ShapeDtypeStruct((_T, _HP), jnp.int32),
    )(x)


def _tc_route_body(e_ref, pos_ref, be_ref):
    ids = e_ref[...]
    dn = (((1,), (0,)), ((), ()))
    up_tri = (lax.broadcasted_iota(jnp.int32, (128, 128), 0)
              <= lax.broadcasted_iota(jnp.int32, (128, 128), 1)
              ).astype(jnp.float32)
    lo_tri = (lax.broadcasted_iota(jnp.int32, (64, 64), 0)
              > lax.broadcasted_iota(jnp.int32, (64, 64), 1)
              ).astype(jnp.float32)
    masks = []
    ranks = []
    counts = []
    for e in range(_E):
        m = (ids == e).astype(jnp.float32)
        p = lax.dot_general(m, up_tri, dn, preferred_element_type=jnp.float32)
        t = p[:, 127:128]
        rows_before = lax.dot_general(
            lo_tri, t, dn, preferred_element_type=jnp.float32)
        counts.append(rows_before[63:64, :] + t[63:64, :])
        masks.append(m)
        ranks.append(p + rows_before - 1.0)
    pos = jnp.zeros((64, 128), jnp.float32)
    off = jnp.zeros((1, 1), jnp.float32)
    ends = []
    for e in range(_E):
        padded = jnp.floor((counts[e] + (_BM - 1)) / _BM) * _BM
        pos = pos + masks[e] * (off + ranks[e])
        off = off + padded
        ends.append(off)
    pos_ref[...] = pos.astype(jnp.int32)
    blk = lax.broadcasted_iota(
        jnp.int32, (8, 128), 1).astype(jnp.float32) * _BM
    be = jnp.zeros((8, 128), jnp.float32)
    for e in range(_E):
        be = be + (blk >= ends[e]).astype(jnp.float32)
    be_ref[...] = jnp.minimum(be, _E - 1).astype(jnp.int32)


def _tc_route(e2d):
    return pl.pallas_call(
        _tc_route_body,
        grid=(1,),
        in_specs=[pl.BlockSpec((64, 128), lambda i: (0, 0))],
        out_specs=(pl.BlockSpec((64, 128), lambda i: (0, 0)),
                   pl.BlockSpec((8, 128), lambda i: (0, 0))),
        out_shape=(jax.ShapeDtypeStruct((64, 128), jnp.int32),
                   jax.ShapeDtypeStruct((8, 128), jnp.int32)),
    )(e2d)


def _sc_dispatch(x_packed, pos_sm, w_sm):
    n_chunks = _TPW // _CGD

    @functools.partial(
        pl.kernel,
        mesh=_sc_mesh(),
        out_type=(
            jax.ShapeDtypeStruct((_NPAD, _HP), jnp.int32),
            jax.ShapeDtypeStruct((_NPAD,), jnp.float32),
        ),
        scratch_types=[
            pltpu.VMEM((_CGD,), jnp.int32),
            pltpu.VMEM((_CGD,), jnp.int32),
            pltpu.VMEM((_CGD,), jnp.int32),
            pltpu.VMEM((_CGD,), jnp.int32),
            pltpu.VMEM((_CGD,), jnp.float32),
            pltpu.VMEM((_CGD,), jnp.float32),
            pltpu.VMEM((_CGD,), jnp.float32),
            pltpu.VMEM((_CGD,), jnp.float32),
            pltpu.VMEM((_CGD, _HP), jnp.int32),
            pltpu.VMEM((_CGD, _HP), jnp.int32),
            pltpu.SemaphoreType.DMA,
            pltpu.SemaphoreType.DMA,
            pltpu.SemaphoreType.DMA,
            pltpu.SemaphoreType.DMA,
            pltpu.SemaphoreType.DMA,
            pltpu.SemaphoreType.DMA,
            pltpu.SemaphoreType.DMA,
            pltpu.SemaphoreType.DMA,
        ],
    )
    def k(xp_hbm, pos_hbm, w_hbm, xs_hbm, wp_hbm,
          p0a, p0b, p1a, p1b, w0a, w0b, w1a, w1b, xca, xcb,
          s0a, s0b, s1a, s1b, s2a, s2b, s3a, s3b):
        wid = lax.axis_index("s") * _NC + lax.axis_index("c")
        tok0 = wid * _TPW
        p0 = (p0a, p0b)
        p1 = (p1a, p1b)
        w0 = (w0a, w0b)
        w1 = (w1a, w1b)
        xc = (xca, xcb)
        sx0 = (s0a, s0b)
        sx1 = (s1a, s1b)
        sw0 = (s2a, s2b)
        sw1 = (s3a, s3b)
        hs = [None] * n_chunks
        for c in range(n_chunks):
            p = c & 1
            if c >= 2:
                for h in hs[c - 2]:
                    h.wait()
            off = tok0 + c * _CGD
            pltpu.sync_copy(pos_hbm.at[pl.ds(off, _CGD)], p0[p])
            pltpu.sync_copy(pos_hbm.at[pl.ds(_T + off, _CGD)], p1[p])
            pltpu.sync_copy(w_hbm.at[pl.ds(off, _CGD)], w0[p])
            pltpu.sync_copy(w_hbm.at[pl.ds(_T + off, _CGD)], w1[p])
            pltpu.sync_copy(xp_hbm.at[pl.ds(off, _CGD)], xc[p])
            hs[c] = (
                pltpu.async_copy(xc[p], xs_hbm.at[p0[p]], sx0[p]),
                pltpu.async_copy(xc[p], xs_hbm.at[p1[p]], sx1[p]),
                pltpu.async_copy(w0[p], wp_hbm.at[p0[p]], sw0[p]),
                pltpu.async_copy(w1[p], wp_hbm.at[p1[p]], sw1[p]),
            )
        for c in (n_chunks - 2, n_chunks - 1):
            for h in hs[c]:
                h.wait()

    return k(x_packed, pos_sm, w_sm)


def _tc_gemm_body(be_ref, x_ref, gpw_ref, upw_ref, dpw_ref, wp_ref, y_ref):
    del be_ref
    dn = (((1,), (1,)), ((), ()))
    xi = x_ref[...]
    lo = lax.bitcast_convert_type(xi << 16, jnp.float32).astype(jnp.bfloat16)
    hi = lax.bitcast_convert_type(
        xi & jnp.int32(-65536), jnp.float32).astype(jnp.bfloat16)
    gpw = gpw_ref[0]
    upw = upw_ref[0]
    f32 = jnp.float32
    gp = (lax.dot_general(lo, gpw[:, :_HP], dn, preferred_element_type=f32)
          + lax.dot_general(hi, gpw[:, _HP:], dn, preferred_element_type=f32))
    up = (lax.dot_general(lo, upw[:, :_HP], dn, preferred_element_type=f32)
          + lax.dot_general(hi, upw[:, _HP:], dn, preferred_element_type=f32))
    act = (gp * jax.nn.sigmoid(gp) * up).astype(jnp.bfloat16)
    y = lax.dot_general(act, dpw_ref[0], dn, preferred_element_type=f32)
    y_ref[...] = y * wp_ref[...]


def _tc_grouped_gemm(block_expert, x_sorted, gpw, upw, dpw, w_pad):
    grid_spec = pltpu.PrefetchScalarGridSpec(
        num_scalar_prefetch=1,
        grid=(_NB,),
        in_specs=[
            pl.BlockSpec((_BM, _HP), lambda i, be: (i, 0)),
            pl.BlockSpec((1, _I, _H), lambda i, be: (be[i], 0, 0)),
            pl.BlockSpec((1, _I, _H), lambda i, be: (be[i], 0, 0)),
            pl.BlockSpec((1, _H, _I), lambda i, be: (be[i], 0, 0)),
            pl.BlockSpec((_BM, 1), lambda i, be: (i, 0)),
        ],
        out_specs=pl.BlockSpec((_BM, _H), lambda i, be: (i, 0)),
    )
    return pl.pallas_call(
        _tc_gemm_body,
        grid_spec=grid_spec,
        out_shape=jax.ShapeDtypeStruct((_NPAD, _H), jnp.float32),
    )(block_expert, x_sorted, gpw, upw, dpw, w_pad)


def _sc_combine(y_sorted, pos_sm):
    cc = 8
    n_chunks = _TPW // cc

    @functools.partial(
        pl.kernel,
        mesh=_sc_mesh(),
        out_type=jax.ShapeDtypeStruct((_T, _H), jnp.float32),
        scratch_types=[
            pltpu.VMEM((cc,), jnp.int32),
            pltpu.VMEM((cc,), jnp.int32),
            pltpu.VMEM((cc,), jnp.int32),
            pltpu.VMEM((cc,), jnp.int32),
            pltpu.VMEM((cc, _H), jnp.float32),
            pltpu.VMEM((cc, _H), jnp.float32),
            pltpu.VMEM((cc, _H), jnp.float32),
            pltpu.VMEM((cc, _H), jnp.float32),
            pltpu.SemaphoreType.DMA,
            pltpu.SemaphoreType.DMA,
            pltpu.SemaphoreType.DMA,
            pltpu.SemaphoreType.DMA,
            pltpu.SemaphoreType.DMA,
            pltpu.SemaphoreType.DMA,
        ],
    )
    def k(y_hbm, pos_hbm, out_hbm, i0a, i0b, i1a, i1b, aa, ab, ba, bb,
          g0a, g0b, g1a, g1b, wsa, wsb):
        wid = lax.axis_index("s") * _NC + lax.axis_index("c")
        i0 = (i0a, i0b)
        i1 = (i1a, i1b)
        av = (aa, ab)
        bv = (ba, bb)
        g0 = (g0a, g0b)
        g1 = (g1a, g1b)
        ws = (wsa, wsb)
        gh = [None] * n_chunks
        wh = [None] * n_chunks

        def add_write(c):
            q = c & 1
            gh[c][0].wait()
            gh[c][1].wait()

            def add_col(j, carry):
                for r in range(cc):
                    sl = pl.ds(j * 16, 16)
                    av[q][r, sl] = av[q][r, sl] + bv[q][r, sl]
                return carry

            lax.fori_loop(0, _H // 16, add_col, 0)
            off = wid * _TPW + c * cc
            wh[c] = pltpu.async_copy(
                av[q], out_hbm.at[pl.ds(off, cc)], ws[q])

        for c in range(n_chunks):
            p = c & 1
            if c >= 2:
                wh[c - 2].wait()
            off = wid * _TPW + c * cc
            pltpu.sync_copy(pos_hbm.at[pl.ds(off, cc)], i0[p])
            pltpu.sync_copy(pos_hbm.at[pl.ds(_T + off, cc)], i1[p])
            gh[c] = (pltpu.async_copy(y_hbm.at[i0[p]], av[p], g0[p]),
                     pltpu.async_copy(y_hbm.at[i1[p]], bv[p], g1[p]))
            if c >= 1:
                add_write(c - 1)
        add_write(n_chunks - 1)
        wh[n_chunks - 2].wait()
        wh[n_chunks - 1].wait()

    return k(y_sorted, pos_sm)


def kernel(hidden_states, gate_w, gate_proj_w, up_proj_w, down_proj_w):
    b, s, h = hidden_states.shape
    x = hidden_states.reshape(-1, h)

    router_logits = x @ gate_w.T
    routing_weights = jax.nn.softmax(router_logits.astype(jnp.float32), axis=1)
    rw_topk, selected_experts = jax.lax.top_k(routing_weights, _K)
    rw_topk = rw_topk / jnp.sum(rw_topk, axis=-1, keepdims=True)
    e_sm = selected_experts.astype(jnp.int32).T.reshape(64, 128)
    w_sm = rw_topk.T.reshape(_NSL)

    x_packed = _tc_pack(x)
    pos2d, be2d = _tc_route(e_sm)
    pos_sm = pos2d.reshape(_NSL)
    block_expert = be2d.reshape(-1)[:_NB]
    x_sorted, w_pad = _sc_dispatch(x_packed, pos_sm, w_sm)
    y_sorted = _tc_grouped_gemm(
        block_expert, x_sorted,
        gate_proj_w.astype(jnp.bfloat16),
        up_proj_w.astype(jnp.bfloat16),
        down_proj_w.astype(jnp.bfloat16),
        w_pad.reshape(_NPAD, 1))
    out = _sc_combine(y_sorted, pos_sm)
    return out.reshape(b, s, h)
